# Initial kernel scaffold; baseline (speedup 1.0000x reference)
import jax
import jax.numpy as jnp
from jax.experimental import pallas as pl


def kernel(node_embs, edge_embs, edge_index, cep_embs, W_O1, b_O1, W_I1, b_I1, W_rel1, b_rel1, W_O2, b_O2, W_I2, b_I2, W_rel2, b_rel2, W_cep, b_cep):
    raise NotImplementedError("write your pallas kernel here")


# speedup vs baseline: 9.3565x; 9.3565x over previous
import functools

import jax
import jax.numpy as jnp
from jax import lax
from jax.experimental import pallas as pl
from jax.experimental.pallas import tpu as pltpu
from jax.experimental.pallas import tpu_sc as plsc

N_NODES = 10000
N_EDGES = 320000
EMB = 128
HID = 32

NT = 16
CH = 128
NCH = 160
EPT = NCH * CH
E_PAD = NT * EPT
N_PAD = 10240
RPT = N_PAD // NT
TRASH = N_PAD - 1

BE = 2000
BR = 2000


def _edge_body(x_ref, wo_ref, wi_ref, wr_ref, brel_ref, out_ref):
    x = x_ref[...]
    yo = jnp.dot(x, wo_ref[...], preferred_element_type=jnp.float32)
    yi = jnp.dot(x, wi_ref[...], preferred_element_type=jnp.float32)
    yr = jnp.maximum(
        jnp.dot(x, wr_ref[...], preferred_element_type=jnp.float32)
        + brel_ref[...], 0.0)
    ones = jnp.ones((x.shape[0], 16), jnp.float32)
    zeros = jnp.zeros((x.shape[0], 16), jnp.float32)
    out_ref[...] = jnp.concatenate([yo, yr, yi, ones, zeros], axis=1)


def _edge_mm(edge_embs, W_O1, W_I1, W_rel1, b_rel1):
    return pl.pallas_call(
        _edge_body,
        grid=(N_EDGES // BE,),
        in_specs=[
            pl.BlockSpec((BE, EMB), lambda i: (i, 0)),
            pl.BlockSpec((EMB, HID), lambda i: (0, 0)),
            pl.BlockSpec((EMB, HID), lambda i: (0, 0)),
            pl.BlockSpec((EMB, HID), lambda i: (0, 0)),
            pl.BlockSpec((1, HID), lambda i: (0, 0)),
        ],
        out_specs=pl.BlockSpec((BE, EMB), lambda i: (i, 0)),
        out_shape=jax.ShapeDtypeStruct((E_PAD, EMB), jnp.float32),
    )(edge_embs, W_O1, W_I1, W_rel1, b_rel1.reshape(1, HID))


def _sc_body(p_hbm, idx_hbm, out_hbm, ibufj, pbuf, acc):
    c = lax.axis_index("c")
    s = lax.axis_index("s")

    zeros16 = jnp.zeros((16,), jnp.float32)

    def _zero_row(i, _):
        for j in range(EMB // 16):
            pbuf[i, pl.ds(j * 16, 16)] = zeros16
        return 0

    lax.fori_loop(0, CH, _zero_row, 0)
    for t in range(RPT // CH):
        pltpu.sync_copy(pbuf, acc.at[pl.ds(s * RPT + t * CH, CH)])
    plsc.subcore_barrier()

    base = s * EPT

    def _chunk(j, _):
        pltpu.sync_copy(idx_hbm.at[c, s, j], ibufj)
        pltpu.sync_copy(p_hbm.at[pl.ds(base + j * CH, CH)], pbuf)
        pltpu.sync_copy(pbuf, acc.at[ibufj], add=True)
        return 0

    lax.fori_loop(0, NCH, _chunk, 0)
    plsc.subcore_barrier()

    for t in range(RPT // CH):
        r0 = s * RPT + t * CH
        pltpu.sync_copy(acc.at[pl.ds(r0, CH)], pbuf)
        pltpu.sync_copy(pbuf, out_hbm.at[c, pl.ds(r0, CH)])


@functools.cache
def _sc_segsum():
    return pl.kernel(
        _sc_body,
        out_type=jax.ShapeDtypeStruct((2, N_PAD, EMB), jnp.float32),
        mesh=plsc.VectorSubcoreMesh(core_axis_name="c", subcore_axis_name="s"),
        scratch_types=[
            pltpu.VMEM((CH,), jnp.int32),
            pltpu.VMEM((CH, EMB), jnp.float32),
            pltpu.VMEM_SHARED((N_PAD, EMB), jnp.float32),
        ],
    )


def _node_body(x_ref, a0_ref, a1_ref,
               wo1_ref, wi1_ref, wo2_ref, wi2_ref, b1_ref, b2_ref, out_ref):
    i = pl.program_id(0)
    x = x_ref[...]
    a0 = a0_ref[...]
    a1 = a1_ref[...]
    deg_d = a0[:, 3 * HID:3 * HID + 1]
    deg_s = a1[:, 3 * HID:3 * HID + 1]
    s_po = a0[:, 0:HID]
    s_hed = a0[:, HID:2 * HID]
    s_pi = a1[:, 2 * HID:3 * HID]
    s_hes = a1[:, HID:2 * HID]
    uo = jnp.dot(x, wo1_ref[...], preferred_element_type=jnp.float32)
    ui = jnp.dot(x, wi1_ref[...], preferred_element_type=jnp.float32)
    hn1 = jnp.maximum(deg_d * uo - s_po + deg_s * ui - s_pi + b1_ref[...], 0.0)
    ho2 = deg_d * hn1 - s_hed
    hi2 = deg_s * hn1 - s_hes
    hn2 = jnp.maximum(
        jnp.dot(ho2, wo2_ref[...], preferred_element_type=jnp.float32)
        + jnp.dot(hi2, wi2_ref[...], preferred_element_type=jnp.float32)
        + b2_ref[...], 0.0)
    part = jnp.sum(hn2, axis=0, keepdims=True)

    @pl.when(i == 0)
    def _():
        out_ref[...] = jnp.zeros_like(out_ref)

    out_ref[...] += part


def _node_phase(node_embs, a0, a1, W_O1, W_I1, W_O2, W_I2, b1, b2):
    return pl.pallas_call(
        _node_body,
        grid=(N_NODES // BR,),
        in_specs=[
            pl.BlockSpec((BR, EMB), lambda i: (i, 0)),
            pl.BlockSpec((BR, EMB), lambda i: (i, 0)),
            pl.BlockSpec((BR, EMB), lambda i: (i, 0)),
            pl.BlockSpec((EMB, HID), lambda i: (0, 0)),
            pl.BlockSpec((EMB, HID), lambda i: (0, 0)),
            pl.BlockSpec((HID, EMB), lambda i: (0, 0)),
            pl.BlockSpec((HID, EMB), lambda i: (0, 0)),
            pl.BlockSpec((1, HID), lambda i: (0, 0)),
            pl.BlockSpec((1, EMB), lambda i: (0, 0)),
        ],
        out_specs=pl.BlockSpec((1, EMB), lambda i: (0, 0)),
        out_shape=jax.ShapeDtypeStruct((1, EMB), jnp.float32),
    )(node_embs, a0, a1, W_O1, W_I1, W_O2, W_I2, b1, b2)


def _head_body(hsum_ref, cep_ref, wc_ref, bc_ref, out_ref):
    cep = jnp.maximum(
        jnp.dot(cep_ref[...], wc_ref[...], preferred_element_type=jnp.float32)
        + bc_ref[...], 0.0)
    hg = hsum_ref[...] * (1.0 / N_NODES)
    out_ref[...] = lax.dot_general(
        hg, cep, (((1,), (1,)), ((), ())),
        preferred_element_type=jnp.float32)


def _head(hsum, cep_embs, W_cep, b_cep):
    n_cep = cep_embs.shape[0]
    return pl.pallas_call(
        _head_body,
        out_shape=jax.ShapeDtypeStruct((1, n_cep), jnp.float32),
    )(hsum, cep_embs, W_cep, b_cep.reshape(1, EMB))


def kernel(node_embs, edge_embs, edge_index, cep_embs,
           W_O1, b_O1, W_I1, b_I1, W_rel1, b_rel1,
           W_O2, b_O2, W_I2, b_I2, W_rel2, b_rel2,
           W_cep, b_cep):
    p = _edge_mm(edge_embs, W_O1, W_I1, W_rel1, b_rel1)
    pad = jnp.full((E_PAD - N_EDGES,), TRASH, jnp.int32)
    idx_all = jnp.stack([jnp.concatenate([edge_index[1], pad]),
                         jnp.concatenate([edge_index[0], pad])])
    idx_all = idx_all.reshape(2, NT, NCH, CH)
    acc = _sc_segsum()(p, idx_all)
    b1 = (b_O1 + b_I1).reshape(1, HID)
    b2 = (b_O2 + b_I2).reshape(1, EMB)
    hsum = _node_phase(node_embs, acc[0], acc[1],
                       W_O1, W_I1, W_O2, W_I2, b1, b2)
    return _head(hsum, cep_embs, W_cep, b_cep)


# speedup vs baseline: 12.4943x; 1.3354x over previous
import functools

import jax
import jax.numpy as jnp
from jax import lax
from jax.experimental import pallas as pl
from jax.experimental.pallas import tpu as pltpu
from jax.experimental.pallas import tpu_sc as plsc

N_NODES = 10000
N_EDGES = 320000
EMB = 128
HID = 32

NT = 16
CH = 128
NCH = 160
EPT = NCH * CH
E_PAD = NT * EPT
N_PAD = 10240
RPT = N_PAD // NT
TRASH = N_PAD - 1

BE = 2000
BR = 2000


def _edge_body(x_ref, wo_ref, wi_ref, wr_ref, brel_ref, out_ref):
    x = x_ref[...]
    yo = jnp.dot(x, wo_ref[...], preferred_element_type=jnp.float32)
    yi = jnp.dot(x, wi_ref[...], preferred_element_type=jnp.float32)
    yr = jnp.maximum(
        jnp.dot(x, wr_ref[...], preferred_element_type=jnp.float32)
        + brel_ref[...], 0.0)
    ones = jnp.ones((x.shape[0], 16), jnp.float32)
    zeros = jnp.zeros((x.shape[0], 16), jnp.float32)
    out_ref[...] = jnp.concatenate([yo, yr, yi, ones, zeros], axis=1)


def _edge_mm(edge_embs, W_O1, W_I1, W_rel1, b_rel1):
    return pl.pallas_call(
        _edge_body,
        grid=(N_EDGES // BE,),
        in_specs=[
            pl.BlockSpec((BE, EMB), lambda i: (i, 0)),
            pl.BlockSpec((EMB, HID), lambda i: (0, 0)),
            pl.BlockSpec((EMB, HID), lambda i: (0, 0)),
            pl.BlockSpec((EMB, HID), lambda i: (0, 0)),
            pl.BlockSpec((1, HID), lambda i: (0, 0)),
        ],
        out_specs=pl.BlockSpec((BE, EMB), lambda i: (i, 0)),
        out_shape=jax.ShapeDtypeStruct((E_PAD, EMB), jnp.float32),
    )(edge_embs, W_O1, W_I1, W_rel1, b_rel1.reshape(1, HID))


def _sc_body(p_hbm, idx_hbm, out_hbm, ibuf0, ibuf1, pbuf0, pbuf1, acc,
             sp0, sp1, si0, si1, ss0, ss1):
    c = lax.axis_index("c")
    s = lax.axis_index("s")

    zeros16 = jnp.zeros((16,), jnp.float32)

    def _zero_row(i, _):
        for j in range(EMB // 16):
            pbuf0[i, pl.ds(j * 16, 16)] = zeros16
        return 0

    lax.fori_loop(0, CH, _zero_row, 0)
    for t in range(RPT // CH):
        pltpu.sync_copy(pbuf0, acc.at[pl.ds(s * RPT + t * CH, CH)])
    plsc.subcore_barrier()

    base = s * EPT

    def load(j, pb, ib, sp, si):
        pltpu.async_copy(idx_hbm.at[c, s, j], ib, si)
        pltpu.async_copy(p_hbm.at[pl.ds(base + j * CH, CH)], pb, sp)

    def wait_load(pb, ib, sp, si):
        pltpu.make_async_copy(idx_hbm.at[c, s, 0], ib, si).wait()
        pltpu.make_async_copy(p_hbm.at[pl.ds(0, CH)], pb, sp).wait()

    load(0, pbuf0, ibuf0, sp0, si0)

    def _g(g, _):
        e = 2 * g
        wait_load(pbuf0, ibuf0, sp0, si0)
        pltpu.async_copy(pbuf0, acc.at[ibuf0], ss0, add=True)

        @pl.when(g > 0)
        def _():
            pltpu.make_async_copy(pbuf1, acc.at[ibuf1], ss1).wait()

        load(e + 1, pbuf1, ibuf1, sp1, si1)
        wait_load(pbuf1, ibuf1, sp1, si1)
        pltpu.async_copy(pbuf1, acc.at[ibuf1], ss1, add=True)
        pltpu.make_async_copy(pbuf0, acc.at[ibuf0], ss0).wait()

        @pl.when(g < NCH // 2 - 1)
        def _():
            load(e + 2, pbuf0, ibuf0, sp0, si0)

        return 0

    lax.fori_loop(0, NCH // 2, _g, 0)
    pltpu.make_async_copy(pbuf1, acc.at[ibuf1], ss1).wait()
    plsc.subcore_barrier()

    for t in range(RPT // CH):
        r0 = s * RPT + t * CH
        pltpu.sync_copy(acc.at[pl.ds(r0, CH)], pbuf0)
        pltpu.sync_copy(pbuf0, out_hbm.at[c, pl.ds(r0, CH)])


@functools.cache
def _sc_segsum():
    return pl.kernel(
        _sc_body,
        out_type=jax.ShapeDtypeStruct((2, N_PAD, EMB), jnp.float32),
        mesh=plsc.VectorSubcoreMesh(core_axis_name="c", subcore_axis_name="s"),
        scratch_types=[
            pltpu.VMEM((CH,), jnp.int32),
            pltpu.VMEM((CH,), jnp.int32),
            pltpu.VMEM((CH, EMB), jnp.float32),
            pltpu.VMEM((CH, EMB), jnp.float32),
            pltpu.VMEM_SHARED((N_PAD, EMB), jnp.float32),
            pltpu.SemaphoreType.DMA,
            pltpu.SemaphoreType.DMA,
            pltpu.SemaphoreType.DMA,
            pltpu.SemaphoreType.DMA,
            pltpu.SemaphoreType.DMA,
            pltpu.SemaphoreType.DMA,
        ],
    )


def _node_body(x_ref, a0_ref, a1_ref,
               wo1_ref, wi1_ref, wo2_ref, wi2_ref, b1_ref, b2_ref, out_ref):
    i = pl.program_id(0)
    x = x_ref[...]
    a0 = a0_ref[...]
    a1 = a1_ref[...]
    deg_d = a0[:, 3 * HID:3 * HID + 1]
    deg_s = a1[:, 3 * HID:3 * HID + 1]
    s_po = a0[:, 0:HID]
    s_hed = a0[:, HID:2 * HID]
    s_pi = a1[:, 2 * HID:3 * HID]
    s_hes = a1[:, HID:2 * HID]
    uo = jnp.dot(x, wo1_ref[...], preferred_element_type=jnp.float32)
    ui = jnp.dot(x, wi1_ref[...], preferred_element_type=jnp.float32)
    hn1 = jnp.maximum(deg_d * uo - s_po + deg_s * ui - s_pi + b1_ref[...], 0.0)
    ho2 = deg_d * hn1 - s_hed
    hi2 = deg_s * hn1 - s_hes
    hn2 = jnp.maximum(
        jnp.dot(ho2, wo2_ref[...], preferred_element_type=jnp.float32)
        + jnp.dot(hi2, wi2_ref[...], preferred_element_type=jnp.float32)
        + b2_ref[...], 0.0)
    part = jnp.sum(hn2, axis=0, keepdims=True)

    @pl.when(i == 0)
    def _():
        out_ref[...] = jnp.zeros_like(out_ref)

    out_ref[...] += part


def _node_phase(node_embs, a0, a1, W_O1, W_I1, W_O2, W_I2, b1, b2):
    return pl.pallas_call(
        _node_body,
        grid=(N_NODES // BR,),
        in_specs=[
            pl.BlockSpec((BR, EMB), lambda i: (i, 0)),
            pl.BlockSpec((BR, EMB), lambda i: (i, 0)),
            pl.BlockSpec((BR, EMB), lambda i: (i, 0)),
            pl.BlockSpec((EMB, HID), lambda i: (0, 0)),
            pl.BlockSpec((EMB, HID), lambda i: (0, 0)),
            pl.BlockSpec((HID, EMB), lambda i: (0, 0)),
            pl.BlockSpec((HID, EMB), lambda i: (0, 0)),
            pl.BlockSpec((1, HID), lambda i: (0, 0)),
            pl.BlockSpec((1, EMB), lambda i: (0, 0)),
        ],
        out_specs=pl.BlockSpec((1, EMB), lambda i: (0, 0)),
        out_shape=jax.ShapeDtypeStruct((1, EMB), jnp.float32),
    )(node_embs, a0, a1, W_O1, W_I1, W_O2, W_I2, b1, b2)


def _head_body(hsum_ref, cep_ref, wc_ref, bc_ref, out_ref):
    cep = jnp.maximum(
        jnp.dot(cep_ref[...], wc_ref[...], preferred_element_type=jnp.float32)
        + bc_ref[...], 0.0)
    hg = hsum_ref[...] * (1.0 / N_NODES)
    out_ref[...] = lax.dot_general(
        hg, cep, (((1,), (1,)), ((), ())),
        preferred_element_type=jnp.float32)


def _head(hsum, cep_embs, W_cep, b_cep):
    n_cep = cep_embs.shape[0]
    return pl.pallas_call(
        _head_body,
        out_shape=jax.ShapeDtypeStruct((1, n_cep), jnp.float32),
    )(hsum, cep_embs, W_cep, b_cep.reshape(1, EMB))


def kernel(node_embs, edge_embs, edge_index, cep_embs,
           W_O1, b_O1, W_I1, b_I1, W_rel1, b_rel1,
           W_O2, b_O2, W_I2, b_I2, W_rel2, b_rel2,
           W_cep, b_cep):
    p = _edge_mm(edge_embs, W_O1, W_I1, W_rel1, b_rel1)
    pad = jnp.full((E_PAD - N_EDGES,), TRASH, jnp.int32)
    idx_all = jnp.stack([jnp.concatenate([edge_index[1], pad]),
                         jnp.concatenate([edge_index[0], pad])])
    idx_all = idx_all.reshape(2, NT, NCH, CH)
    acc = _sc_segsum()(p, idx_all)
    b1 = (b_O1 + b_I1).reshape(1, HID)
    b2 = (b_O2 + b_I2).reshape(1, EMB)
    hsum = _node_phase(node_embs, acc[0], acc[1],
                       W_O1, W_I1, W_O2, W_I2, b1, b2)
    return _head(hsum, cep_embs, W_cep, b_cep)


# speedup vs baseline: 15.0842x; 1.2073x over previous
import functools

import jax
import jax.numpy as jnp
from jax import lax
from jax.experimental import pallas as pl
from jax.experimental.pallas import tpu as pltpu
from jax.experimental.pallas import tpu_sc as plsc

N_NODES = 10000
N_EDGES = 320000
EMB = 128
HID = 32

NT = 16
CH = 128
NCH = 160
EPT = NCH * CH
E_PAD = NT * EPT
N_PAD = 10240
RPT = N_PAD // NT
TRASH = N_PAD - 1

BE = 8000
BR = 2000


def _edge_body(x_ref, wo_ref, wi_ref, wr_ref, brel_ref, out_ref):
    x = x_ref[...]
    yo = jnp.dot(x, wo_ref[...], preferred_element_type=jnp.float32)
    yi = jnp.dot(x, wi_ref[...], preferred_element_type=jnp.float32)
    yr = jnp.maximum(
        jnp.dot(x, wr_ref[...], preferred_element_type=jnp.float32)
        + brel_ref[...], 0.0)
    ones = jnp.ones((x.shape[0], 16), jnp.float32)
    zeros = jnp.zeros((x.shape[0], 16), jnp.float32)
    out_ref[...] = jnp.concatenate([yo, yr, yi, ones, zeros], axis=1)


def _edge_mm(edge_embs, W_O1, W_I1, W_rel1, b_rel1):
    return pl.pallas_call(
        _edge_body,
        grid=(N_EDGES // BE,),
        in_specs=[
            pl.BlockSpec((BE, EMB), lambda i: (i, 0)),
            pl.BlockSpec((EMB, HID), lambda i: (0, 0)),
            pl.BlockSpec((EMB, HID), lambda i: (0, 0)),
            pl.BlockSpec((EMB, HID), lambda i: (0, 0)),
            pl.BlockSpec((1, HID), lambda i: (0, 0)),
        ],
        out_specs=pl.BlockSpec((BE, EMB), lambda i: (i, 0)),
        out_shape=jax.ShapeDtypeStruct((E_PAD, EMB), jnp.float32),
    )(edge_embs, W_O1, W_I1, W_rel1, b_rel1.reshape(1, HID))


def _sc_body(p_hbm, idx_hbm, out_hbm, ibuf0, ibuf1, pbuf0, pbuf1, acc,
             sp0, sp1, si0, si1, ss0, ss1):
    c = lax.axis_index("c")
    s = lax.axis_index("s")

    zeros16 = jnp.zeros((16,), jnp.float32)

    def _zero_row(i, _):
        for j in range(EMB // 16):
            pbuf0[i, pl.ds(j * 16, 16)] = zeros16
        return 0

    lax.fori_loop(0, CH, _zero_row, 0)
    for t in range(RPT // CH):
        pltpu.sync_copy(pbuf0, acc.at[pl.ds(s * RPT + t * CH, CH)])
    plsc.subcore_barrier()

    base = s * EPT

    def load(j, pb, ib, sp, si):
        pltpu.async_copy(idx_hbm.at[c, s, j], ib, si)
        pltpu.async_copy(p_hbm.at[pl.ds(base + j * CH, CH)], pb, sp)

    def wait_load(pb, ib, sp, si):
        pltpu.make_async_copy(idx_hbm.at[c, s, 0], ib, si).wait()
        pltpu.make_async_copy(p_hbm.at[pl.ds(0, CH)], pb, sp).wait()

    load(0, pbuf0, ibuf0, sp0, si0)

    def _g(g, _):
        e = 2 * g
        wait_load(pbuf0, ibuf0, sp0, si0)
        pltpu.async_copy(pbuf0, acc.at[ibuf0], ss0, add=True)

        @pl.when(g > 0)
        def _():
            pltpu.make_async_copy(pbuf1, acc.at[ibuf1], ss1).wait()

        load(e + 1, pbuf1, ibuf1, sp1, si1)
        wait_load(pbuf1, ibuf1, sp1, si1)
        pltpu.async_copy(pbuf1, acc.at[ibuf1], ss1, add=True)
        pltpu.make_async_copy(pbuf0, acc.at[ibuf0], ss0).wait()

        @pl.when(g < NCH // 2 - 1)
        def _():
            load(e + 2, pbuf0, ibuf0, sp0, si0)

        return 0

    lax.fori_loop(0, NCH // 2, _g, 0)
    pltpu.make_async_copy(pbuf1, acc.at[ibuf1], ss1).wait()
    plsc.subcore_barrier()

    for t in range(RPT // CH):
        r0 = s * RPT + t * CH
        pltpu.sync_copy(acc.at[pl.ds(r0, CH)], pbuf0)
        pltpu.sync_copy(pbuf0, out_hbm.at[c, pl.ds(r0, CH)])


@functools.cache
def _sc_segsum():
    return pl.kernel(
        _sc_body,
        out_type=jax.ShapeDtypeStruct((2, N_PAD, EMB), jnp.float32),
        mesh=plsc.VectorSubcoreMesh(core_axis_name="c", subcore_axis_name="s"),
        scratch_types=[
            pltpu.VMEM((CH,), jnp.int32),
            pltpu.VMEM((CH,), jnp.int32),
            pltpu.VMEM((CH, EMB), jnp.float32),
            pltpu.VMEM((CH, EMB), jnp.float32),
            pltpu.VMEM_SHARED((N_PAD, EMB), jnp.float32),
            pltpu.SemaphoreType.DMA,
            pltpu.SemaphoreType.DMA,
            pltpu.SemaphoreType.DMA,
            pltpu.SemaphoreType.DMA,
            pltpu.SemaphoreType.DMA,
            pltpu.SemaphoreType.DMA,
        ],
    )


def _node_body(x_ref, a0_ref, a1_ref,
               wo1_ref, wi1_ref, wo2_ref, wi2_ref, b1_ref, b2_ref, out_ref):
    i = pl.program_id(0)
    x = x_ref[...]
    a0 = a0_ref[0]
    a1 = a1_ref[0]
    deg_d = a0[:, 3 * HID:3 * HID + 1]
    deg_s = a1[:, 3 * HID:3 * HID + 1]
    s_po = a0[:, 0:HID]
    s_hed = a0[:, HID:2 * HID]
    s_pi = a1[:, 2 * HID:3 * HID]
    s_hes = a1[:, HID:2 * HID]
    uo = jnp.dot(x, wo1_ref[...], preferred_element_type=jnp.float32)
    ui = jnp.dot(x, wi1_ref[...], preferred_element_type=jnp.float32)
    hn1 = jnp.maximum(deg_d * uo - s_po + deg_s * ui - s_pi + b1_ref[...], 0.0)
    ho2 = deg_d * hn1 - s_hed
    hi2 = deg_s * hn1 - s_hes
    hn2 = jnp.maximum(
        jnp.dot(ho2, wo2_ref[...], preferred_element_type=jnp.float32)
        + jnp.dot(hi2, wi2_ref[...], preferred_element_type=jnp.float32)
        + b2_ref[...], 0.0)
    part = jnp.sum(hn2, axis=0, keepdims=True)

    @pl.when(i == 0)
    def _():
        out_ref[...] = jnp.zeros_like(out_ref)

    out_ref[...] += part


def _node_phase(node_embs, acc, W_O1, W_I1, W_O2, W_I2, b1, b2):
    return pl.pallas_call(
        _node_body,
        grid=(N_NODES // BR,),
        in_specs=[
            pl.BlockSpec((BR, EMB), lambda i: (i, 0)),
            pl.BlockSpec((1, BR, EMB), lambda i: (0, i, 0)),
            pl.BlockSpec((1, BR, EMB), lambda i: (1, i, 0)),
            pl.BlockSpec((EMB, HID), lambda i: (0, 0)),
            pl.BlockSpec((EMB, HID), lambda i: (0, 0)),
            pl.BlockSpec((HID, EMB), lambda i: (0, 0)),
            pl.BlockSpec((HID, EMB), lambda i: (0, 0)),
            pl.BlockSpec((1, HID), lambda i: (0, 0)),
            pl.BlockSpec((1, EMB), lambda i: (0, 0)),
        ],
        out_specs=pl.BlockSpec((1, EMB), lambda i: (0, 0)),
        out_shape=jax.ShapeDtypeStruct((1, EMB), jnp.float32),
    )(node_embs, acc, acc, W_O1, W_I1, W_O2, W_I2, b1, b2)


def _head_body(hsum_ref, cep_ref, wc_ref, bc_ref, out_ref):
    cep = jnp.maximum(
        jnp.dot(cep_ref[...], wc_ref[...], preferred_element_type=jnp.float32)
        + bc_ref[...], 0.0)
    hg = hsum_ref[...] * (1.0 / N_NODES)
    out_ref[...] = lax.dot_general(
        hg, cep, (((1,), (1,)), ((), ())),
        preferred_element_type=jnp.float32)


def _head(hsum, cep_embs, W_cep, b_cep):
    n_cep = cep_embs.shape[0]
    return pl.pallas_call(
        _head_body,
        out_shape=jax.ShapeDtypeStruct((1, n_cep), jnp.float32),
    )(hsum, cep_embs, W_cep, b_cep.reshape(1, EMB))


def kernel(node_embs, edge_embs, edge_index, cep_embs,
           W_O1, b_O1, W_I1, b_I1, W_rel1, b_rel1,
           W_O2, b_O2, W_I2, b_I2, W_rel2, b_rel2,
           W_cep, b_cep):
    p = _edge_mm(edge_embs, W_O1, W_I1, W_rel1, b_rel1)
    pad = jnp.full((E_PAD - N_EDGES,), TRASH, jnp.int32)
    idx_all = jnp.stack([jnp.concatenate([edge_index[1], pad]),
                         jnp.concatenate([edge_index[0], pad])])
    idx_all = idx_all.reshape(2, NT, NCH, CH)
    acc = _sc_segsum()(p, idx_all)
    b1 = (b_O1 + b_I1).reshape(1, HID)
    b2 = (b_O2 + b_I2).reshape(1, EMB)
    hsum = _node_phase(node_embs, acc,
                       W_O1, W_I1, W_O2, W_I2, b1, b2)
    return _head(hsum, cep_embs, W_cep, b_cep)


# speedup vs baseline: 15.4952x; 1.0273x over previous
import functools

import jax
import jax.numpy as jnp
from jax import lax
from jax.experimental import pallas as pl
from jax.experimental.pallas import tpu as pltpu
from jax.experimental.pallas import tpu_sc as plsc

N_NODES = 10000
N_EDGES = 320000
EMB = 128
HID = 32

NT = 16
CH = 128
NCH = 160
EPT = NCH * CH
E_PAD = NT * EPT
N_PAD = 10240
RPT = N_PAD // NT
TRASH = N_PAD - 1

BE = 16000
BR = 10000


def _edge_body(x_ref, wo_ref, wi_ref, wr_ref, brel_ref, out_ref):
    x = x_ref[...]
    yo = jnp.dot(x, wo_ref[...], preferred_element_type=jnp.float32)
    yi = jnp.dot(x, wi_ref[...], preferred_element_type=jnp.float32)
    yr = jnp.maximum(
        jnp.dot(x, wr_ref[...], preferred_element_type=jnp.float32)
        + brel_ref[...], 0.0)
    ones = jnp.ones((x.shape[0], 16), jnp.float32)
    zeros = jnp.zeros((x.shape[0], 16), jnp.float32)
    out_ref[...] = jnp.concatenate([yo, yr, yi, ones, zeros], axis=1)


def _edge_mm(edge_embs, W_O1, W_I1, W_rel1, b_rel1):
    return pl.pallas_call(
        _edge_body,
        grid=(N_EDGES // BE,),
        in_specs=[
            pl.BlockSpec((BE, EMB), lambda i: (i, 0)),
            pl.BlockSpec((EMB, HID), lambda i: (0, 0)),
            pl.BlockSpec((EMB, HID), lambda i: (0, 0)),
            pl.BlockSpec((EMB, HID), lambda i: (0, 0)),
            pl.BlockSpec((1, HID), lambda i: (0, 0)),
        ],
        out_specs=pl.BlockSpec((BE, EMB), lambda i: (i, 0)),
        out_shape=jax.ShapeDtypeStruct((E_PAD, EMB), jnp.float32),
    )(edge_embs, W_O1, W_I1, W_rel1, b_rel1.reshape(1, HID))


def _sc_body(p_hbm, idx_hbm, out_hbm, ibuf0, ibuf1, pbuf0, pbuf1, acc,
             sp0, sp1, si0, si1, ss0, ss1):
    c = lax.axis_index("c")
    s = lax.axis_index("s")

    zeros16 = jnp.zeros((16,), jnp.float32)

    def _zero_row(i, _):
        for j in range(EMB // 16):
            pbuf0[i, pl.ds(j * 16, 16)] = zeros16
        return 0

    lax.fori_loop(0, CH, _zero_row, 0)
    for t in range(RPT // CH):
        pltpu.sync_copy(pbuf0, acc.at[pl.ds(s * RPT + t * CH, CH)])
    plsc.subcore_barrier()

    base = s * EPT

    def load(j, pb, ib, sp, si):
        pltpu.async_copy(idx_hbm.at[c, s, j], ib, si)
        pltpu.async_copy(p_hbm.at[pl.ds(base + j * CH, CH)], pb, sp)

    def wait_load(pb, ib, sp, si):
        pltpu.make_async_copy(idx_hbm.at[c, s, 0], ib, si).wait()
        pltpu.make_async_copy(p_hbm.at[pl.ds(0, CH)], pb, sp).wait()

    load(0, pbuf0, ibuf0, sp0, si0)

    def _g(g, _):
        e = 2 * g
        wait_load(pbuf0, ibuf0, sp0, si0)
        pltpu.async_copy(pbuf0, acc.at[ibuf0], ss0, add=True)

        @pl.when(g > 0)
        def _():
            pltpu.make_async_copy(pbuf1, acc.at[ibuf1], ss1).wait()

        load(e + 1, pbuf1, ibuf1, sp1, si1)
        wait_load(pbuf1, ibuf1, sp1, si1)
        pltpu.async_copy(pbuf1, acc.at[ibuf1], ss1, add=True)
        pltpu.make_async_copy(pbuf0, acc.at[ibuf0], ss0).wait()

        @pl.when(g < NCH // 2 - 1)
        def _():
            load(e + 2, pbuf0, ibuf0, sp0, si0)

        return 0

    lax.fori_loop(0, NCH // 2, _g, 0)
    pltpu.make_async_copy(pbuf1, acc.at[ibuf1], ss1).wait()
    plsc.subcore_barrier()

    for t in range(RPT // CH):
        r0 = s * RPT + t * CH
        pltpu.sync_copy(acc.at[pl.ds(r0, CH)], pbuf0)
        pltpu.sync_copy(pbuf0, out_hbm.at[c, pl.ds(r0, CH)])


@functools.cache
def _sc_segsum():
    return pl.kernel(
        _sc_body,
        out_type=jax.ShapeDtypeStruct((2, N_PAD, EMB), jnp.float32),
        mesh=plsc.VectorSubcoreMesh(core_axis_name="c", subcore_axis_name="s"),
        scratch_types=[
            pltpu.VMEM((CH,), jnp.int32),
            pltpu.VMEM((CH,), jnp.int32),
            pltpu.VMEM((CH, EMB), jnp.float32),
            pltpu.VMEM((CH, EMB), jnp.float32),
            pltpu.VMEM_SHARED((N_PAD, EMB), jnp.float32),
            pltpu.SemaphoreType.DMA,
            pltpu.SemaphoreType.DMA,
            pltpu.SemaphoreType.DMA,
            pltpu.SemaphoreType.DMA,
            pltpu.SemaphoreType.DMA,
            pltpu.SemaphoreType.DMA,
        ],
    )


def _node_body(x_ref, a0_ref, a1_ref,
               wo1_ref, wi1_ref, wo2_ref, wi2_ref, b1_ref, b2_ref, out_ref):
    i = pl.program_id(0)
    x = x_ref[...]
    a0 = a0_ref[0]
    a1 = a1_ref[0]
    deg_d = a0[:, 3 * HID:3 * HID + 1]
    deg_s = a1[:, 3 * HID:3 * HID + 1]
    s_po = a0[:, 0:HID]
    s_hed = a0[:, HID:2 * HID]
    s_pi = a1[:, 2 * HID:3 * HID]
    s_hes = a1[:, HID:2 * HID]
    uo = jnp.dot(x, wo1_ref[...], preferred_element_type=jnp.float32)
    ui = jnp.dot(x, wi1_ref[...], preferred_element_type=jnp.float32)
    hn1 = jnp.maximum(deg_d * uo - s_po + deg_s * ui - s_pi + b1_ref[...], 0.0)
    ho2 = deg_d * hn1 - s_hed
    hi2 = deg_s * hn1 - s_hes
    hn2 = jnp.maximum(
        jnp.dot(ho2, wo2_ref[...], preferred_element_type=jnp.float32)
        + jnp.dot(hi2, wi2_ref[...], preferred_element_type=jnp.float32)
        + b2_ref[...], 0.0)
    part = jnp.sum(hn2, axis=0, keepdims=True)

    @pl.when(i == 0)
    def _():
        out_ref[...] = jnp.zeros_like(out_ref)

    out_ref[...] += part


def _node_phase(node_embs, acc, W_O1, W_I1, W_O2, W_I2, b1, b2):
    return pl.pallas_call(
        _node_body,
        grid=(N_NODES // BR,),
        in_specs=[
            pl.BlockSpec((BR, EMB), lambda i: (i, 0)),
            pl.BlockSpec((1, BR, EMB), lambda i: (0, i, 0)),
            pl.BlockSpec((1, BR, EMB), lambda i: (1, i, 0)),
            pl.BlockSpec((EMB, HID), lambda i: (0, 0)),
            pl.BlockSpec((EMB, HID), lambda i: (0, 0)),
            pl.BlockSpec((HID, EMB), lambda i: (0, 0)),
            pl.BlockSpec((HID, EMB), lambda i: (0, 0)),
            pl.BlockSpec((1, HID), lambda i: (0, 0)),
            pl.BlockSpec((1, EMB), lambda i: (0, 0)),
        ],
        out_specs=pl.BlockSpec((1, EMB), lambda i: (0, 0)),
        out_shape=jax.ShapeDtypeStruct((1, EMB), jnp.float32),
    )(node_embs, acc, acc, W_O1, W_I1, W_O2, W_I2, b1, b2)


def _head_body(hsum_ref, cep_ref, wc_ref, bc_ref, out_ref):
    cep = jnp.maximum(
        jnp.dot(cep_ref[...], wc_ref[...], preferred_element_type=jnp.float32)
        + bc_ref[...], 0.0)
    hg = hsum_ref[...] * (1.0 / N_NODES)
    out_ref[...] = lax.dot_general(
        hg, cep, (((1,), (1,)), ((), ())),
        preferred_element_type=jnp.float32)


def _head(hsum, cep_embs, W_cep, b_cep):
    n_cep = cep_embs.shape[0]
    return pl.pallas_call(
        _head_body,
        out_shape=jax.ShapeDtypeStruct((1, n_cep), jnp.float32),
    )(hsum, cep_embs, W_cep, b_cep.reshape(1, EMB))


def kernel(node_embs, edge_embs, edge_index, cep_embs,
           W_O1, b_O1, W_I1, b_I1, W_rel1, b_rel1,
           W_O2, b_O2, W_I2, b_I2, W_rel2, b_rel2,
           W_cep, b_cep):
    p = _edge_mm(edge_embs, W_O1, W_I1, W_rel1, b_rel1)
    pad = jnp.full((E_PAD - N_EDGES,), TRASH, jnp.int32)
    idx_all = jnp.stack([jnp.concatenate([edge_index[1], pad]),
                         jnp.concatenate([edge_index[0], pad])])
    idx_all = idx_all.reshape(2, NT, NCH, CH)
    acc = _sc_segsum()(p, idx_all)
    b1 = (b_O1 + b_I1).reshape(1, HID)
    b2 = (b_O2 + b_I2).reshape(1, EMB)
    hsum = _node_phase(node_embs, acc,
                       W_O1, W_I1, W_O2, W_I2, b1, b2)
    return _head(hsum, cep_embs, W_cep, b_cep)


# speedup vs baseline: 15.7574x; 1.0169x over previous
import functools

import jax
import jax.numpy as jnp
from jax import lax
from jax.experimental import pallas as pl
from jax.experimental.pallas import tpu as pltpu
from jax.experimental.pallas import tpu_sc as plsc

N_NODES = 10000
N_EDGES = 320000
EMB = 128
HID = 32

NT = 16
CH = 128
NCH = 80
EPT = NCH * CH
E_HALF = N_EDGES // 2
E_PAD = NT * EPT
N_PAD = 10240
RPT = N_PAD // NT
TRASH = N_PAD - 1

BE = 16000
BR = 10000


def _edge_body(x_ref, wo_ref, wi_ref, wr_ref, brel_ref, out_ref):
    x = x_ref[...]
    yo = jnp.dot(x, wo_ref[...], preferred_element_type=jnp.float32)
    yi = jnp.dot(x, wi_ref[...], preferred_element_type=jnp.float32)
    yr = jnp.maximum(
        jnp.dot(x, wr_ref[...], preferred_element_type=jnp.float32)
        + brel_ref[...], 0.0)
    ones = jnp.ones((x.shape[0], 16), jnp.float32)
    zeros = jnp.zeros((x.shape[0], 16), jnp.float32)
    out_ref[...] = jnp.concatenate([yo, yr, yi, ones, zeros], axis=1)


def _edge_mm(edge_embs, W_O1, W_I1, W_rel1, b_rel1, half):
    off = half * (E_HALF // BE)
    return pl.pallas_call(
        _edge_body,
        grid=(E_HALF // BE,),
        in_specs=[
            pl.BlockSpec((BE, EMB), lambda i: (i + off, 0)),
            pl.BlockSpec((EMB, HID), lambda i: (0, 0)),
            pl.BlockSpec((EMB, HID), lambda i: (0, 0)),
            pl.BlockSpec((EMB, HID), lambda i: (0, 0)),
            pl.BlockSpec((1, HID), lambda i: (0, 0)),
        ],
        out_specs=pl.BlockSpec((BE, EMB), lambda i: (i, 0)),
        out_shape=jax.ShapeDtypeStruct((E_PAD, EMB), jnp.float32),
    )(edge_embs, W_O1, W_I1, W_rel1, b_rel1.reshape(1, HID))


def _sc_body(p_hbm, idx_hbm, out_hbm, ibuf0, ibuf1, pbuf0, pbuf1, acc,
             sp0, sp1, si0, si1, ss0, ss1):
    c = lax.axis_index("c")
    s = lax.axis_index("s")

    zeros16 = jnp.zeros((16,), jnp.float32)

    def _zero_row(i, _):
        for j in range(EMB // 16):
            pbuf0[i, pl.ds(j * 16, 16)] = zeros16
        return 0

    lax.fori_loop(0, CH, _zero_row, 0)
    for t in range(RPT // CH):
        pltpu.sync_copy(pbuf0, acc.at[pl.ds(s * RPT + t * CH, CH)])
    plsc.subcore_barrier()

    base = s * EPT

    def load(j, pb, ib, sp, si):
        pltpu.async_copy(idx_hbm.at[c, s, j], ib, si)
        pltpu.async_copy(p_hbm.at[pl.ds(base + j * CH, CH)], pb, sp)

    def wait_load(pb, ib, sp, si):
        pltpu.make_async_copy(idx_hbm.at[c, s, 0], ib, si).wait()
        pltpu.make_async_copy(p_hbm.at[pl.ds(0, CH)], pb, sp).wait()

    load(0, pbuf0, ibuf0, sp0, si0)

    def _g(g, _):
        e = 2 * g
        wait_load(pbuf0, ibuf0, sp0, si0)
        pltpu.async_copy(pbuf0, acc.at[ibuf0], ss0, add=True)

        @pl.when(g > 0)
        def _():
            pltpu.make_async_copy(pbuf1, acc.at[ibuf1], ss1).wait()

        load(e + 1, pbuf1, ibuf1, sp1, si1)
        wait_load(pbuf1, ibuf1, sp1, si1)
        pltpu.async_copy(pbuf1, acc.at[ibuf1], ss1, add=True)
        pltpu.make_async_copy(pbuf0, acc.at[ibuf0], ss0).wait()

        @pl.when(g < NCH // 2 - 1)
        def _():
            load(e + 2, pbuf0, ibuf0, sp0, si0)

        return 0

    lax.fori_loop(0, NCH // 2, _g, 0)
    pltpu.make_async_copy(pbuf1, acc.at[ibuf1], ss1).wait()
    plsc.subcore_barrier()

    for t in range(RPT // CH):
        r0 = s * RPT + t * CH
        pltpu.sync_copy(acc.at[pl.ds(r0, CH)], pbuf0)
        pltpu.sync_copy(pbuf0, out_hbm.at[c, pl.ds(r0, CH)])


@functools.cache
def _sc_segsum():
    return pl.kernel(
        _sc_body,
        out_type=jax.ShapeDtypeStruct((2, N_PAD, EMB), jnp.float32),
        mesh=plsc.VectorSubcoreMesh(core_axis_name="c", subcore_axis_name="s"),
        scratch_types=[
            pltpu.VMEM((CH,), jnp.int32),
            pltpu.VMEM((CH,), jnp.int32),
            pltpu.VMEM((CH, EMB), jnp.float32),
            pltpu.VMEM((CH, EMB), jnp.float32),
            pltpu.VMEM_SHARED((N_PAD, EMB), jnp.float32),
            pltpu.SemaphoreType.DMA,
            pltpu.SemaphoreType.DMA,
            pltpu.SemaphoreType.DMA,
            pltpu.SemaphoreType.DMA,
            pltpu.SemaphoreType.DMA,
            pltpu.SemaphoreType.DMA,
        ],
    )


def _node_body(x_ref, a0a_ref, a1a_ref, a0b_ref, a1b_ref,
               wo1_ref, wi1_ref, wo2_ref, wi2_ref, b1_ref, b2_ref, out_ref):
    i = pl.program_id(0)
    x = x_ref[...]
    a0 = a0a_ref[0] + a0b_ref[0]
    a1 = a1a_ref[0] + a1b_ref[0]
    deg_d = a0[:, 3 * HID:3 * HID + 1]
    deg_s = a1[:, 3 * HID:3 * HID + 1]
    s_po = a0[:, 0:HID]
    s_hed = a0[:, HID:2 * HID]
    s_pi = a1[:, 2 * HID:3 * HID]
    s_hes = a1[:, HID:2 * HID]
    uo = jnp.dot(x, wo1_ref[...], preferred_element_type=jnp.float32)
    ui = jnp.dot(x, wi1_ref[...], preferred_element_type=jnp.float32)
    hn1 = jnp.maximum(deg_d * uo - s_po + deg_s * ui - s_pi + b1_ref[...], 0.0)
    ho2 = deg_d * hn1 - s_hed
    hi2 = deg_s * hn1 - s_hes
    hn2 = jnp.maximum(
        jnp.dot(ho2, wo2_ref[...], preferred_element_type=jnp.float32)
        + jnp.dot(hi2, wi2_ref[...], preferred_element_type=jnp.float32)
        + b2_ref[...], 0.0)
    part = jnp.sum(hn2, axis=0, keepdims=True)

    @pl.when(i == 0)
    def _():
        out_ref[...] = jnp.zeros_like(out_ref)

    out_ref[...] += part


def _node_phase(node_embs, acca, accb, W_O1, W_I1, W_O2, W_I2, b1, b2):
    return pl.pallas_call(
        _node_body,
        grid=(N_NODES // BR,),
        in_specs=[
            pl.BlockSpec((BR, EMB), lambda i: (i, 0)),
            pl.BlockSpec((1, BR, EMB), lambda i: (0, i, 0)),
            pl.BlockSpec((1, BR, EMB), lambda i: (1, i, 0)),
            pl.BlockSpec((1, BR, EMB), lambda i: (0, i, 0)),
            pl.BlockSpec((1, BR, EMB), lambda i: (1, i, 0)),
            pl.BlockSpec((EMB, HID), lambda i: (0, 0)),
            pl.BlockSpec((EMB, HID), lambda i: (0, 0)),
            pl.BlockSpec((HID, EMB), lambda i: (0, 0)),
            pl.BlockSpec((HID, EMB), lambda i: (0, 0)),
            pl.BlockSpec((1, HID), lambda i: (0, 0)),
            pl.BlockSpec((1, EMB), lambda i: (0, 0)),
        ],
        out_specs=pl.BlockSpec((1, EMB), lambda i: (0, 0)),
        out_shape=jax.ShapeDtypeStruct((1, EMB), jnp.float32),
    )(node_embs, acca, acca, accb, accb, W_O1, W_I1, W_O2, W_I2, b1, b2)


def _head_body(hsum_ref, cep_ref, wc_ref, bc_ref, out_ref):
    cep = jnp.maximum(
        jnp.dot(cep_ref[...], wc_ref[...], preferred_element_type=jnp.float32)
        + bc_ref[...], 0.0)
    hg = hsum_ref[...] * (1.0 / N_NODES)
    out_ref[...] = lax.dot_general(
        hg, cep, (((1,), (1,)), ((), ())),
        preferred_element_type=jnp.float32)


def _head(hsum, cep_embs, W_cep, b_cep):
    n_cep = cep_embs.shape[0]
    return pl.pallas_call(
        _head_body,
        out_shape=jax.ShapeDtypeStruct((1, n_cep), jnp.float32),
    )(hsum, cep_embs, W_cep, b_cep.reshape(1, EMB))


def kernel(node_embs, edge_embs, edge_index, cep_embs,
           W_O1, b_O1, W_I1, b_I1, W_rel1, b_rel1,
           W_O2, b_O2, W_I2, b_I2, W_rel2, b_rel2,
           W_cep, b_cep):
    pa = _edge_mm(edge_embs, W_O1, W_I1, W_rel1, b_rel1, 0)
    pb = _edge_mm(edge_embs, W_O1, W_I1, W_rel1, b_rel1, 1)
    pad = jnp.full((E_PAD - E_HALF,), TRASH, jnp.int32)

    def half_idx(h):
        dst = lax.dynamic_slice(edge_index[1], (h * E_HALF,), (E_HALF,))
        src = lax.dynamic_slice(edge_index[0], (h * E_HALF,), (E_HALF,))
        ia = jnp.stack([jnp.concatenate([dst, pad]),
                        jnp.concatenate([src, pad])])
        return ia.reshape(2, NT, NCH, CH)

    acca = _sc_segsum()(pa, half_idx(0))
    accb = _sc_segsum()(pb, half_idx(1))
    b1 = (b_O1 + b_I1).reshape(1, HID)
    b2 = (b_O2 + b_I2).reshape(1, EMB)
    hsum = _node_phase(node_embs, acca, accb,
                       W_O1, W_I1, W_O2, W_I2, b1, b2)
    return _head(hsum, cep_embs, W_cep, b_cep)


# speedup vs baseline: 15.8083x; 1.0032x over previous
import functools

import jax
import jax.numpy as jnp
from jax import lax
from jax.experimental import pallas as pl
from jax.experimental.pallas import tpu as pltpu
from jax.experimental.pallas import tpu_sc as plsc

N_NODES = 10000
N_EDGES = 320000
EMB = 128
HID = 32

NT = 16
CH = 128
NCH = 80
EPT = NCH * CH
E_HALF = N_EDGES // 2
E_PAD = NT * EPT
N_PAD = 10240
RPT = N_PAD // NT
TRASH = N_PAD - 1

BE = 16000
BR = 10000


def _edge_body(x_ref, wo_ref, wi_ref, wr_ref, brel_ref, out_ref):
    x = x_ref[...]
    yo = jnp.dot(x, wo_ref[...], preferred_element_type=jnp.float32)
    yi = jnp.dot(x, wi_ref[...], preferred_element_type=jnp.float32)
    yr = jnp.maximum(
        jnp.dot(x, wr_ref[...], preferred_element_type=jnp.float32)
        + brel_ref[...], 0.0)
    ones = jnp.ones((x.shape[0], 16), jnp.float32)
    zeros = jnp.zeros((x.shape[0], 16), jnp.float32)
    out_ref[...] = jnp.concatenate([yo, yr, yi, ones, zeros], axis=1)


def _edge_mm(edge_embs, W_O1, W_I1, W_rel1, b_rel1, half):
    off = half * (E_HALF // BE)
    return pl.pallas_call(
        _edge_body,
        grid=(E_HALF // BE,),
        in_specs=[
            pl.BlockSpec((BE, EMB), lambda i: (i + off, 0)),
            pl.BlockSpec((EMB, HID), lambda i: (0, 0)),
            pl.BlockSpec((EMB, HID), lambda i: (0, 0)),
            pl.BlockSpec((EMB, HID), lambda i: (0, 0)),
            pl.BlockSpec((1, HID), lambda i: (0, 0)),
        ],
        out_specs=pl.BlockSpec((BE, EMB), lambda i: (i, 0)),
        out_shape=jax.ShapeDtypeStruct((E_PAD, EMB), jnp.float32),
    )(edge_embs, W_O1, W_I1, W_rel1, b_rel1.reshape(1, HID))


def _sc_body(p_hbm, idx_hbm, out_hbm, ibuf0, ibuf1, pbuf0, pbuf1, acc,
             sp0, sp1, si0, si1, ss0, ss1):
    c = lax.axis_index("c")
    s = lax.axis_index("s")

    zeros16 = jnp.zeros((16,), jnp.float32)

    def _zero_row(i, _):
        for j in range(EMB // 16):
            pbuf0[i, pl.ds(j * 16, 16)] = zeros16
        return 0

    lax.fori_loop(0, CH, _zero_row, 0)
    for t in range(RPT // CH):
        pltpu.async_copy(pbuf0, acc.at[pl.ds(s * RPT + t * CH, CH)], sp1)
    for t in range(RPT // CH):
        pltpu.make_async_copy(pbuf0, acc.at[pl.ds(0, CH)], sp1).wait()
    plsc.subcore_barrier()

    base = s * EPT

    def load(j, pb, ib, sp, si):
        pltpu.async_copy(idx_hbm.at[c, s, j], ib, si)
        pltpu.async_copy(p_hbm.at[pl.ds(base + j * CH, CH)], pb, sp)

    def wait_load(pb, ib, sp, si):
        pltpu.make_async_copy(idx_hbm.at[c, s, 0], ib, si).wait()
        pltpu.make_async_copy(p_hbm.at[pl.ds(0, CH)], pb, sp).wait()

    load(0, pbuf0, ibuf0, sp0, si0)

    def _g(g, _):
        e = 2 * g
        wait_load(pbuf0, ibuf0, sp0, si0)
        pltpu.async_copy(pbuf0, acc.at[ibuf0], ss0, add=True)

        @pl.when(g > 0)
        def _():
            pltpu.make_async_copy(pbuf1, acc.at[ibuf1], ss1).wait()

        load(e + 1, pbuf1, ibuf1, sp1, si1)
        wait_load(pbuf1, ibuf1, sp1, si1)
        pltpu.async_copy(pbuf1, acc.at[ibuf1], ss1, add=True)
        pltpu.make_async_copy(pbuf0, acc.at[ibuf0], ss0).wait()

        @pl.when(g < NCH // 2 - 1)
        def _():
            load(e + 2, pbuf0, ibuf0, sp0, si0)

        return 0

    lax.fori_loop(0, NCH // 2, _g, 0)
    pltpu.make_async_copy(pbuf1, acc.at[ibuf1], ss1).wait()
    plsc.subcore_barrier()

    for t in range(RPT // CH):
        r0 = s * RPT + t * CH
        pltpu.async_copy(acc.at[pl.ds(r0, CH)], out_hbm.at[c, pl.ds(r0, CH)],
                         sp0)
    for t in range(RPT // CH):
        pltpu.make_async_copy(acc.at[pl.ds(0, CH)], out_hbm.at[c, pl.ds(0, CH)],
                              sp0).wait()


@functools.cache
def _sc_segsum():
    return pl.kernel(
        _sc_body,
        out_type=jax.ShapeDtypeStruct((2, N_PAD, EMB), jnp.float32),
        mesh=plsc.VectorSubcoreMesh(core_axis_name="c", subcore_axis_name="s"),
        scratch_types=[
            pltpu.VMEM((CH,), jnp.int32),
            pltpu.VMEM((CH,), jnp.int32),
            pltpu.VMEM((CH, EMB), jnp.float32),
            pltpu.VMEM((CH, EMB), jnp.float32),
            pltpu.VMEM_SHARED((N_PAD, EMB), jnp.float32),
            pltpu.SemaphoreType.DMA,
            pltpu.SemaphoreType.DMA,
            pltpu.SemaphoreType.DMA,
            pltpu.SemaphoreType.DMA,
            pltpu.SemaphoreType.DMA,
            pltpu.SemaphoreType.DMA,
        ],
    )


def _node_body(x_ref, a0a_ref, a1a_ref, a0b_ref, a1b_ref,
               wo1_ref, wi1_ref, wo2_ref, wi2_ref, b1_ref, b2_ref, out_ref):
    i = pl.program_id(0)
    x = x_ref[...]
    a0 = a0a_ref[0] + a0b_ref[0]
    a1 = a1a_ref[0] + a1b_ref[0]
    deg_d = a0[:, 3 * HID:3 * HID + 1]
    deg_s = a1[:, 3 * HID:3 * HID + 1]
    s_po = a0[:, 0:HID]
    s_hed = a0[:, HID:2 * HID]
    s_pi = a1[:, 2 * HID:3 * HID]
    s_hes = a1[:, HID:2 * HID]
    uo = jnp.dot(x, wo1_ref[...], preferred_element_type=jnp.float32)
    ui = jnp.dot(x, wi1_ref[...], preferred_element_type=jnp.float32)
    hn1 = jnp.maximum(deg_d * uo - s_po + deg_s * ui - s_pi + b1_ref[...], 0.0)
    ho2 = deg_d * hn1 - s_hed
    hi2 = deg_s * hn1 - s_hes
    hn2 = jnp.maximum(
        jnp.dot(ho2, wo2_ref[...], preferred_element_type=jnp.float32)
        + jnp.dot(hi2, wi2_ref[...], preferred_element_type=jnp.float32)
        + b2_ref[...], 0.0)
    part = jnp.sum(hn2, axis=0, keepdims=True)

    @pl.when(i == 0)
    def _():
        out_ref[...] = jnp.zeros_like(out_ref)

    out_ref[...] += part


def _node_phase(node_embs, acca, accb, W_O1, W_I1, W_O2, W_I2, b1, b2):
    return pl.pallas_call(
        _node_body,
        grid=(N_NODES // BR,),
        in_specs=[
            pl.BlockSpec((BR, EMB), lambda i: (i, 0)),
            pl.BlockSpec((1, BR, EMB), lambda i: (0, i, 0)),
            pl.BlockSpec((1, BR, EMB), lambda i: (1, i, 0)),
            pl.BlockSpec((1, BR, EMB), lambda i: (0, i, 0)),
            pl.BlockSpec((1, BR, EMB), lambda i: (1, i, 0)),
            pl.BlockSpec((EMB, HID), lambda i: (0, 0)),
            pl.BlockSpec((EMB, HID), lambda i: (0, 0)),
            pl.BlockSpec((HID, EMB), lambda i: (0, 0)),
            pl.BlockSpec((HID, EMB), lambda i: (0, 0)),
            pl.BlockSpec((1, HID), lambda i: (0, 0)),
            pl.BlockSpec((1, EMB), lambda i: (0, 0)),
        ],
        out_specs=pl.BlockSpec((1, EMB), lambda i: (0, 0)),
        out_shape=jax.ShapeDtypeStruct((1, EMB), jnp.float32),
    )(node_embs, acca, acca, accb, accb, W_O1, W_I1, W_O2, W_I2, b1, b2)


def _head_body(hsum_ref, cep_ref, wc_ref, bc_ref, out_ref):
    cep = jnp.maximum(
        jnp.dot(cep_ref[...], wc_ref[...], preferred_element_type=jnp.float32)
        + bc_ref[...], 0.0)
    hg = hsum_ref[...] * (1.0 / N_NODES)
    out_ref[...] = lax.dot_general(
        hg, cep, (((1,), (1,)), ((), ())),
        preferred_element_type=jnp.float32)


def _head(hsum, cep_embs, W_cep, b_cep):
    n_cep = cep_embs.shape[0]
    return pl.pallas_call(
        _head_body,
        out_shape=jax.ShapeDtypeStruct((1, n_cep), jnp.float32),
    )(hsum, cep_embs, W_cep, b_cep.reshape(1, EMB))


def kernel(node_embs, edge_embs, edge_index, cep_embs,
           W_O1, b_O1, W_I1, b_I1, W_rel1, b_rel1,
           W_O2, b_O2, W_I2, b_I2, W_rel2, b_rel2,
           W_cep, b_cep):
    pa = _edge_mm(edge_embs, W_O1, W_I1, W_rel1, b_rel1, 0)
    pb = _edge_mm(edge_embs, W_O1, W_I1, W_rel1, b_rel1, 1)
    pad = jnp.full((E_PAD - E_HALF,), TRASH, jnp.int32)

    def half_idx(h):
        dst = lax.dynamic_slice(edge_index[1], (h * E_HALF,), (E_HALF,))
        src = lax.dynamic_slice(edge_index[0], (h * E_HALF,), (E_HALF,))
        ia = jnp.stack([jnp.concatenate([dst, pad]),
                        jnp.concatenate([src, pad])])
        return ia.reshape(2, NT, NCH, CH)

    acca = _sc_segsum()(pa, half_idx(0))
    accb = _sc_segsum()(pb, half_idx(1))
    b1 = (b_O1 + b_I1).reshape(1, HID)
    b2 = (b_O2 + b_I2).reshape(1, EMB)
    hsum = _node_phase(node_embs, acca, accb,
                       W_O1, W_I1, W_O2, W_I2, b1, b2)
    return _head(hsum, cep_embs, W_cep, b_cep)
